# Initial kernel scaffold; baseline (speedup 1.0000x reference)
import functools

import jax
import jax.numpy as jnp
from jax import lax
from jax.experimental import pallas as pl
from jax.experimental.pallas import tpu as pltpu
from jax.experimental.pallas import tpu_sc as plsc

N_NODES = 100000
N_EDGES = 3200000
F = 64
NPAD = 100352
RNG = NPAD // 4
DUMMY = 16
ACC_ROWS = RNG + DUMMY
NC, NS, L = 2, 16, 16
NW = NC * NS

_MESH = dict(core_axis_name="c", subcore_axis_name="s")


def _wid():
    return lax.axis_index("s") * NC + lax.axis_index("c")


def _share(total, w):
    q, rem = total // NW, total % NW
    start = w * q + jnp.minimum(w, rem)
    cnt = q + jnp.where(w < rem, 1, 0)
    return start, cnt


def _bincount_body(row2d, zrow, out, idx2d, ones2d, acc):
    c = lax.axis_index("c")
    s = lax.axis_index("s")
    w = _wid()
    pltpu.sync_copy(zrow, acc.at[pl.ds(s * (NPAD // NS), NPAD // NS)])
    for k in range(8):
        for t in range(8):
            ones2d[k, pl.ds(16 * t, 16)] = jnp.full((16,), 1.0, jnp.float32)
    plsc.subcore_barrier()
    start, cnt = _share(N_EDGES // 1024, w)

    def chunk(ci, carry):
        pltpu.sync_copy(row2d.at[pl.ds(ci * 8, 8)], idx2d)
        pltpu.sync_copy(ones2d, acc.at[idx2d], add=True)
        return carry

    lax.fori_loop(start, start + cnt, chunk, 0)
    plsc.subcore_barrier()
    pltpu.sync_copy(acc.at[pl.ds(s * (NPAD // NS), NPAD // NS)],
                    out.at[pl.ds(c * NPAD + s * (NPAD // NS), NPAD // NS)])


_bincount = functools.partial(
    pl.kernel, _bincount_body,
    out_type=jax.ShapeDtypeStruct((2 * NPAD,), jnp.float32),
    mesh=plsc.VectorSubcoreMesh(**_MESH),
    scratch_types=[
        pltpu.VMEM((8, 128), jnp.int32),
        pltpu.VMEM((8, 128), jnp.float32),
        pltpu.VMEM_SHARED((NPAD,), jnp.float32),
    ],
)()


EC = 2000
NCH = N_EDGES // EC


def _agg_body(row_h, col_h, g_h, zacc, out, row_v, col_v, stg_r, stg_c,
              crow, ccol, rows_v, acc):
    c = lax.axis_index("c")
    s = lax.axis_index("s")
    iotav = lax.iota(jnp.int32, 16)
    dmy_r = RNG + iotav
    dmy_c = iotav * 613 + s * 37

    for p in range(2):
        win = 2 * c + p
        lo = win * RNG
        pltpu.sync_copy(zacc, acc.at[pl.ds(s * (ACC_ROWS // NS), ACC_ROWS // NS)])
        plsc.subcore_barrier()

        def do_row(nrn):
            n, rn = nrn
            for k in range(8):
                crow[rn, pl.ds(16 * k, 16)] = stg_r[pl.ds(16 * k, 16)]
                ccol[rn, pl.ds(16 * k, 16)] = stg_c[pl.ds(16 * k, 16)]
            stg_r[pl.ds(0, 16)] = stg_r[pl.ds(128, 16)]
            stg_c[pl.ds(0, 16)] = stg_c[pl.ds(128, 16)]
            rn2 = rn + 1

            def fire(r):
                pltpu.sync_copy(g_h.at[ccol], rows_v)
                pltpu.sync_copy(rows_v, acc.at[crow], add=True)
                return 0

            rn3 = lax.cond(rn2 == 8, fire, lambda r: r, rn2)
            return (n - 128, rn3)

        def vreg(j, carry):
            n, rn = carry
            r = row_v[pl.ds(j * 16, 16)]
            cc = col_v[pl.ds(j * 16, 16)]
            m = (r >= lo) & (r < lo + RNG)
            plsc.store_compressed(stg_r.at[pl.ds(n, 16)], r - lo, mask=m)
            plsc.store_compressed(stg_c.at[pl.ds(n, 16)], cc, mask=m)
            n = n + jnp.sum(m.astype(jnp.int32))
            return lax.cond(n >= 128, do_row, lambda x: x, (n, rn))

        def chunk(ci, carry):
            base = (ci * NS + s) * EC
            pltpu.sync_copy(row_h.at[pl.ds(base, EC)], row_v)
            pltpu.sync_copy(col_h.at[pl.ds(base, EC)], col_v)
            return lax.fori_loop(0, EC // 16, vreg, carry)

        n, rn = lax.fori_loop(0, NCH // NS, chunk, (0, 0))

        for k in range(9):
            off = jnp.minimum(n + 16 * k, 128)
            stg_r[pl.ds(off, 16)] = dmy_r
            stg_c[pl.ds(off, 16)] = dmy_c
        for rr in range(8):
            @pl.when(rr == rn)
            def _():
                for k in range(8):
                    crow[rr, pl.ds(16 * k, 16)] = stg_r[pl.ds(16 * k, 16)]
                    ccol[rr, pl.ds(16 * k, 16)] = stg_c[pl.ds(16 * k, 16)]

            @pl.when(rr > rn)
            def _():
                for k in range(8):
                    crow[rr, pl.ds(16 * k, 16)] = dmy_r
                    ccol[rr, pl.ds(16 * k, 16)] = dmy_c
        pltpu.sync_copy(g_h.at[ccol], rows_v)
        pltpu.sync_copy(rows_v, acc.at[crow], add=True)
        plsc.subcore_barrier()
        pltpu.sync_copy(acc.at[pl.ds(s * (RNG // NS), RNG // NS)],
                        out.at[pl.ds(lo + s * (RNG // NS), RNG // NS)])
        plsc.subcore_barrier()


_agg = functools.partial(
    pl.kernel, _agg_body,
    out_type=jax.ShapeDtypeStruct((NPAD, F), jnp.float32),
    mesh=plsc.VectorSubcoreMesh(**_MESH),
    scratch_types=[
        pltpu.VMEM((EC,), jnp.int32),
        pltpu.VMEM((EC,), jnp.int32),
        pltpu.VMEM((144,), jnp.int32),
        pltpu.VMEM((144,), jnp.int32),
        pltpu.VMEM((8, 128), jnp.int32),
        pltpu.VMEM((8, 128), jnp.int32),
        pltpu.VMEM((8, 128, F), jnp.float32),
        pltpu.VMEM_SHARED((ACC_ROWS, F), jnp.float32),
    ],
)()


GC = 512
NLCH = 2 * N_EDGES // GC


def _logits_body(src2d, dst2d, z_h, out, sidx, didx, zs, zd, tbuf, lbuf):
    w = _wid()
    iot16 = lax.iota(jnp.int32, 16) * 16
    start, cnt = _share(NLCH, w)

    def chunk(ci, carry):
        pltpu.sync_copy(src2d.at[pl.ds(ci * 4, 4)], sidx)
        pltpu.sync_copy(dst2d.at[pl.ds(ci * 4, 4)], didx)
        pltpu.sync_copy(z_h.at[sidx], zs)
        pltpu.sync_copy(z_h.at[didx], zd)

        def group(g, carry2):
            for e in range(16):
                flat = g * 16 + e
                rhi = flat // 128
                rlo = lax.rem(flat, 128)
                acc = zs[rhi, rlo, pl.ds(0, 16)] * zd[rhi, rlo, pl.ds(0, 16)]
                for k in range(1, 4):
                    acc = acc + (zs[rhi, rlo, pl.ds(16 * k, 16)]
                                 * zd[rhi, rlo, pl.ds(16 * k, 16)])
                plsc.store_scatter(tbuf, [iot16 + e], acc)
            res = tbuf[pl.ds(0, 16)]
            for k in range(1, 16):
                res = res + tbuf[pl.ds(16 * k, 16)]
            lbuf[pl.ds(g * 16, 16)] = res
            return carry2

        lax.fori_loop(0, GC // 16, group, 0)
        pltpu.sync_copy(lbuf, out.at[pl.ds(ci * GC, GC)])
        return carry

    lax.fori_loop(start, start + cnt, chunk, 0)


_logits = functools.partial(
    pl.kernel, _logits_body,
    out_type=jax.ShapeDtypeStruct((2 * N_EDGES,), jnp.float32),
    mesh=plsc.VectorSubcoreMesh(**_MESH),
    scratch_types=[
        pltpu.VMEM((4, 128), jnp.int32),
        pltpu.VMEM((4, 128), jnp.int32),
        pltpu.VMEM((4, 128, F), jnp.float32),
        pltpu.VMEM((4, 128, F), jnp.float32),
        pltpu.VMEM((256,), jnp.float32),
        pltpu.VMEM((GC,), jnp.float32),
    ],
)()


BN = 2048
GRID = NPAD // BN


def _mm1_body(da_ref, db_ref, x_ref, w_ref, b_ref, g_ref, dis_ref):
    deg = da_ref[...] + db_ref[...]
    dis = jnp.where(deg > 0, lax.rsqrt(deg), 0.0)
    h = jnp.dot(x_ref[...], w_ref[...],
                preferred_element_type=jnp.float32) + b_ref[...]
    g_ref[...] = dis[:, None] * h
    dis_ref[...] = dis


def _mm1(da, db, xp, w1, b1):
    return pl.pallas_call(
        _mm1_body,
        grid=(GRID,),
        in_specs=[
            pl.BlockSpec((BN,), lambda i: (i,)),
            pl.BlockSpec((BN,), lambda i: (i,)),
            pl.BlockSpec((BN, 128), lambda i: (i, 0)),
            pl.BlockSpec((128, F), lambda i: (0, 0)),
            pl.BlockSpec((1, F), lambda i: (0, 0)),
        ],
        out_specs=[
            pl.BlockSpec((BN, F), lambda i: (i, 0)),
            pl.BlockSpec((BN,), lambda i: (i,)),
        ],
        out_shape=[
            jax.ShapeDtypeStruct((NPAD, F), jnp.float32),
            jax.ShapeDtypeStruct((NPAD,), jnp.float32),
        ],
    )(da, db, xp, w1, b1)


def _mm2_body(dis_ref, a_ref, w_ref, b_ref, g_ref):
    dis = dis_ref[...]
    h1 = jnp.maximum(dis[:, None] * a_ref[...], 0.0)
    g_ref[...] = dis[:, None] * (
        jnp.dot(h1, w_ref[...], preferred_element_type=jnp.float32) + b_ref[...])


def _mm2(dis, a1, w2, b2):
    return pl.pallas_call(
        _mm2_body,
        grid=(GRID,),
        in_specs=[
            pl.BlockSpec((BN,), lambda i: (i,)),
            pl.BlockSpec((BN, F), lambda i: (i, 0)),
            pl.BlockSpec((F, F), lambda i: (0, 0)),
            pl.BlockSpec((1, F), lambda i: (0, 0)),
        ],
        out_specs=pl.BlockSpec((BN, F), lambda i: (i, 0)),
        out_shape=jax.ShapeDtypeStruct((NPAD, F), jnp.float32),
    )(dis, a1, w2, b2)


def _scale_body(dis_ref, a_ref, z_ref):
    z_ref[...] = dis_ref[...][:, None] * a_ref[...]


def _scale(dis, a2):
    return pl.pallas_call(
        _scale_body,
        grid=(GRID,),
        in_specs=[
            pl.BlockSpec((BN,), lambda i: (i,)),
            pl.BlockSpec((BN, F), lambda i: (i, 0)),
        ],
        out_specs=pl.BlockSpec((BN, F), lambda i: (i, 0)),
        out_shape=jax.ShapeDtypeStruct((NPAD, F), jnp.float32),
    )(dis, a2)


def kernel(x, pos_edge_index, neg_edge_index, W1, b1, W2, b2):
    pe = pos_edge_index.astype(jnp.int32)
    ne = neg_edge_index.astype(jnp.int32)
    row, col = pe[0], pe[1]
    row2d = row.reshape(N_EDGES // 128, 128)
    xp = jnp.pad(x, ((0, NPAD - N_NODES), (0, 0)))
    zrow = jnp.zeros((NPAD // NS,), jnp.float32)
    zacc = jnp.zeros((ACC_ROWS // NS, F), jnp.float32)

    deg2 = _bincount(row2d, zrow)
    g1, dis = _mm1(deg2[:NPAD], deg2[NPAD:], xp, W1, b1.reshape(1, F))
    a1 = _agg(row, col, g1, zacc)
    g2 = _mm2(dis, a1, W2, b2.reshape(1, F))
    a2 = _agg(row, col, g2, zacc)
    z = _scale(dis, a2)

    src2d = jnp.concatenate([pe[0], ne[0]]).reshape(-1, 128)
    dst2d = jnp.concatenate([pe[1], ne[1]]).reshape(-1, 128)
    return _logits(src2d, dst2d, z)


# speedup vs baseline: 10.0636x; 10.0636x over previous
import functools

import jax
import jax.numpy as jnp
from jax import lax
from jax.experimental import pallas as pl
from jax.experimental.pallas import tpu as pltpu
from jax.experimental.pallas import tpu_sc as plsc

N_NODES = 100000
N_EDGES = 3200000
F = 64
NPAD = 100352
RNG = NPAD // 8
NC, NS, L = 2, 16, 16
NW = NC * NS

_MESH = dict(core_axis_name="c", subcore_axis_name="s")
_SC_PARAMS = pltpu.CompilerParams(use_tc_tiling_on_sc=False,
                                  needs_layout_passes=False)


def _wid():
    return lax.axis_index("s") * NC + lax.axis_index("c")


def _share(total, idx, parts):
    q, rem = total // parts, total % parts
    start = idx * q + jnp.minimum(idx, rem)
    cnt = q + jnp.where(idx < rem, 1, 0)
    return start, cnt


BCC = 1024


def _bincount_body(row_h, zrow, out, idx_v, ones_v, acc):
    c = lax.axis_index("c")
    s = lax.axis_index("s")
    pltpu.sync_copy(zrow, acc.at[pl.ds(s * (NPAD // NS), NPAD // NS)])
    for t in range(BCC // 16):
        ones_v[pl.ds(16 * t, 16)] = jnp.full((16,), 1.0, jnp.float32)
    plsc.subcore_barrier()
    start, cnt = _share(N_EDGES // BCC, _wid(), NW)

    def chunk(ci, carry):
        pltpu.sync_copy(row_h.at[pl.ds(ci * BCC, BCC)], idx_v)
        pltpu.sync_copy(ones_v, acc.at[idx_v], add=True)
        return carry

    lax.fori_loop(start, start + cnt, chunk, 0)
    plsc.subcore_barrier()
    pltpu.sync_copy(acc.at[pl.ds(s * (NPAD // NS), NPAD // NS)],
                    out.at[pl.ds(c * NPAD + s * (NPAD // NS), NPAD // NS)])


_bincount = functools.partial(
    pl.kernel, _bincount_body,
    out_type=jax.ShapeDtypeStruct((2 * NPAD,), jnp.float32),
    mesh=plsc.VectorSubcoreMesh(**_MESH),
    scratch_types=[
        pltpu.VMEM((BCC,), jnp.int32),
        pltpu.VMEM((BCC,), jnp.float32),
        pltpu.VMEM_SHARED((NPAD,), jnp.float32),
    ],
    compiler_params=_SC_PARAMS,
)()


EC = 1024
NCH = N_EDGES // EC


def _agg_body(row_h, col_h, g_h, zacc, out, row_v, col_v, mrow, mcol,
              rows_v, acc):
    c = lax.axis_index("c")
    s = lax.axis_index("s")
    start, cnt = _share(NCH, s, NS)

    for p in range(4):
        lo = (4 * c + p) * RNG
        pltpu.sync_copy(zacc, acc.at[pl.ds(s * (RNG // NS), RNG // NS)])
        plsc.subcore_barrier()

        def vreg(j, carry):
            r = row_v[pl.ds(j * 16, 16)]
            cc = col_v[pl.ds(j * 16, 16)]
            m = (r >= lo) & (r < lo + RNG)
            mrow[pl.ds(j * 16, 16)] = jnp.where(m, r - lo, -1)
            mcol[pl.ds(j * 16, 16)] = jnp.where(m, cc, -1)
            return carry

        def chunk(ci, carry):
            pltpu.sync_copy(row_h.at[pl.ds(ci * EC, EC)], row_v)
            pltpu.sync_copy(col_h.at[pl.ds(ci * EC, EC)], col_v)
            lax.fori_loop(0, EC // 16, vreg, 0)
            pltpu.sync_copy(g_h.at[plsc.Indices(mcol, ignored_value=-1)],
                            rows_v)
            pltpu.sync_copy(rows_v,
                            acc.at[plsc.Indices(mrow, ignored_value=-1)],
                            add=True)
            return carry

        lax.fori_loop(start, start + cnt, chunk, 0)
        plsc.subcore_barrier()
        pltpu.sync_copy(acc.at[pl.ds(s * (RNG // NS), RNG // NS)],
                        out.at[pl.ds(lo + s * (RNG // NS), RNG // NS)])
        plsc.subcore_barrier()


_agg = functools.partial(
    pl.kernel, _agg_body,
    out_type=jax.ShapeDtypeStruct((NPAD, F), jnp.float32),
    mesh=plsc.VectorSubcoreMesh(**_MESH),
    scratch_types=[
        pltpu.VMEM((EC,), jnp.int32),
        pltpu.VMEM((EC,), jnp.int32),
        pltpu.VMEM((EC,), jnp.int32),
        pltpu.VMEM((EC,), jnp.int32),
        pltpu.VMEM((EC, F), jnp.float32),
        pltpu.VMEM_SHARED((RNG, F), jnp.float32),
    ],
    compiler_params=_SC_PARAMS,
)()


GC = 512
NLCH = 2 * N_EDGES // GC


def _logits_body(src_h, dst_h, z_h, out, sidx, didx, zs, zd, tbuf, lbuf):
    iot16 = lax.iota(jnp.int32, 16) * 16
    start, cnt = _share(NLCH, _wid(), NW)

    def chunk(ci, carry):
        pltpu.sync_copy(src_h.at[pl.ds(ci * GC, GC)], sidx)
        pltpu.sync_copy(dst_h.at[pl.ds(ci * GC, GC)], didx)
        pltpu.sync_copy(z_h.at[sidx], zs)
        pltpu.sync_copy(z_h.at[didx], zd)

        def group(g, carry2):
            for e in range(16):
                row = g * 16 + e
                acc = zs[row, pl.ds(0, 16)] * zd[row, pl.ds(0, 16)]
                for k in range(1, 4):
                    acc = acc + (zs[row, pl.ds(16 * k, 16)]
                                 * zd[row, pl.ds(16 * k, 16)])
                plsc.store_scatter(tbuf, [iot16 + e], acc)
            res = tbuf[pl.ds(0, 16)]
            for k in range(1, 16):
                res = res + tbuf[pl.ds(16 * k, 16)]
            lbuf[pl.ds(g * 16, 16)] = res
            return carry2

        lax.fori_loop(0, GC // 16, group, 0)
        pltpu.sync_copy(lbuf, out.at[pl.ds(ci * GC, GC)])
        return carry

    lax.fori_loop(start, start + cnt, chunk, 0)


_logits = functools.partial(
    pl.kernel, _logits_body,
    out_type=jax.ShapeDtypeStruct((2 * N_EDGES,), jnp.float32),
    mesh=plsc.VectorSubcoreMesh(**_MESH),
    scratch_types=[
        pltpu.VMEM((GC,), jnp.int32),
        pltpu.VMEM((GC,), jnp.int32),
        pltpu.VMEM((GC, F), jnp.float32),
        pltpu.VMEM((GC, F), jnp.float32),
        pltpu.VMEM((256,), jnp.float32),
        pltpu.VMEM((GC,), jnp.float32),
    ],
    compiler_params=_SC_PARAMS,
)()


BN = 2048
GRID = NPAD // BN


def _mm1_body(da_ref, db_ref, x_ref, w_ref, b_ref, g_ref, dis_ref):
    deg = da_ref[...] + db_ref[...]
    dis = jnp.where(deg > 0, lax.rsqrt(deg), 0.0)
    h = jnp.dot(x_ref[...], w_ref[...],
                preferred_element_type=jnp.float32) + b_ref[...]
    g_ref[...] = dis[:, None] * h
    dis_ref[...] = dis


def _mm1(da, db, xp, w1, b1):
    return pl.pallas_call(
        _mm1_body,
        grid=(GRID,),
        in_specs=[
            pl.BlockSpec((BN,), lambda i: (i,)),
            pl.BlockSpec((BN,), lambda i: (i,)),
            pl.BlockSpec((BN, 128), lambda i: (i, 0)),
            pl.BlockSpec((128, F), lambda i: (0, 0)),
            pl.BlockSpec((1, F), lambda i: (0, 0)),
        ],
        out_specs=[
            pl.BlockSpec((BN, F), lambda i: (i, 0)),
            pl.BlockSpec((BN,), lambda i: (i,)),
        ],
        out_shape=[
            jax.ShapeDtypeStruct((NPAD, F), jnp.float32),
            jax.ShapeDtypeStruct((NPAD,), jnp.float32),
        ],
    )(da, db, xp, w1, b1)


def _mm2_body(dis_ref, a_ref, w_ref, b_ref, g_ref):
    dis = dis_ref[...]
    h1 = jnp.maximum(dis[:, None] * a_ref[...], 0.0)
    g_ref[...] = dis[:, None] * (
        jnp.dot(h1, w_ref[...], preferred_element_type=jnp.float32) + b_ref[...])


def _mm2(dis, a1, w2, b2):
    return pl.pallas_call(
        _mm2_body,
        grid=(GRID,),
        in_specs=[
            pl.BlockSpec((BN,), lambda i: (i,)),
            pl.BlockSpec((BN, F), lambda i: (i, 0)),
            pl.BlockSpec((F, F), lambda i: (0, 0)),
            pl.BlockSpec((1, F), lambda i: (0, 0)),
        ],
        out_specs=pl.BlockSpec((BN, F), lambda i: (i, 0)),
        out_shape=jax.ShapeDtypeStruct((NPAD, F), jnp.float32),
    )(dis, a1, w2, b2)


def _scale_body(dis_ref, a_ref, z_ref):
    z_ref[...] = dis_ref[...][:, None] * a_ref[...]


def _scale(dis, a2):
    return pl.pallas_call(
        _scale_body,
        grid=(GRID,),
        in_specs=[
            pl.BlockSpec((BN,), lambda i: (i,)),
            pl.BlockSpec((BN, F), lambda i: (i, 0)),
        ],
        out_specs=pl.BlockSpec((BN, F), lambda i: (i, 0)),
        out_shape=jax.ShapeDtypeStruct((NPAD, F), jnp.float32),
    )(dis, a2)


def kernel(x, pos_edge_index, neg_edge_index, W1, b1, W2, b2):
    pe = pos_edge_index.astype(jnp.int32)
    ne = neg_edge_index.astype(jnp.int32)
    row, col = pe[0], pe[1]
    xp = jnp.pad(x, ((0, NPAD - N_NODES), (0, 0)))
    zrow = jnp.zeros((NPAD // NS,), jnp.float32)
    zacc = jnp.zeros((RNG // NS, F), jnp.float32)

    deg2 = _bincount(row, zrow)
    g1, dis = _mm1(deg2[:NPAD], deg2[NPAD:], xp, W1, b1.reshape(1, F))
    a1 = _agg(row, col, g1, zacc)
    g2 = _mm2(dis, a1, W2, b2.reshape(1, F))
    a2 = _agg(row, col, g2, zacc)
    z = _scale(dis, a2)

    src = jnp.concatenate([pe[0], ne[0]])
    dst = jnp.concatenate([pe[1], ne[1]])
    return _logits(src, dst, z)


# speedup vs baseline: 16.5593x; 1.6455x over previous
import functools

import jax
import jax.numpy as jnp
from jax import lax
from jax.experimental import pallas as pl
from jax.experimental.pallas import tpu as pltpu
from jax.experimental.pallas import tpu_sc as plsc

N_NODES = 100000
N_EDGES = 3200000
F = 64
NPAD = 100352
RNG = NPAD // 8
NC, NS, L = 2, 16, 16
NW = NC * NS

_MESH = dict(core_axis_name="c", subcore_axis_name="s")
_SC_PARAMS = pltpu.CompilerParams(use_tc_tiling_on_sc=False,
                                  needs_layout_passes=False)


def _wid():
    return lax.axis_index("s") * NC + lax.axis_index("c")


def _share(total, idx, parts):
    q, rem = total // parts, total % parts
    start = idx * q + jnp.minimum(idx, rem)
    cnt = q + jnp.where(idx < rem, 1, 0)
    return start, cnt


BCC = 1024


def _bincount_body(row_h, zrow, out, idx_v, ones_v, acc):
    c = lax.axis_index("c")
    s = lax.axis_index("s")
    pltpu.sync_copy(zrow, acc.at[pl.ds(s * (NPAD // NS), NPAD // NS)])
    for t in range(BCC // 16):
        ones_v[pl.ds(16 * t, 16)] = jnp.full((16,), 1.0, jnp.float32)
    plsc.subcore_barrier()
    start, cnt = _share(N_EDGES // BCC, _wid(), NW)

    def chunk(ci, carry):
        pltpu.sync_copy(row_h.at[pl.ds(ci * BCC, BCC)], idx_v)
        pltpu.sync_copy(ones_v, acc.at[idx_v], add=True)
        return carry

    lax.fori_loop(start, start + cnt, chunk, 0)
    plsc.subcore_barrier()
    pltpu.sync_copy(acc.at[pl.ds(s * (NPAD // NS), NPAD // NS)],
                    out.at[pl.ds(c * NPAD + s * (NPAD // NS), NPAD // NS)])


_bincount = functools.partial(
    pl.kernel, _bincount_body,
    out_type=jax.ShapeDtypeStruct((2 * NPAD,), jnp.float32),
    mesh=plsc.VectorSubcoreMesh(**_MESH),
    scratch_types=[
        pltpu.VMEM((BCC,), jnp.int32),
        pltpu.VMEM((BCC,), jnp.float32),
        pltpu.VMEM_SHARED((NPAD,), jnp.float32),
    ],
    compiler_params=_SC_PARAMS,
)()


EC = 512
NCH = N_EDGES // EC


def _agg_body(row_h, col_h, g_h, zacc, out, row_v, col_v, mrow, mcol,
              rows_v, acc, sem_i, sem_g, sem_s):
    c = lax.axis_index("c")
    s = lax.axis_index("s")
    start, cnt = _share(NCH, s, NS)

    def idx_issue(b, ci):
        pltpu.async_copy(row_h.at[pl.ds(ci * EC, EC)], row_v.at[b],
                         sem_i.at[b])
        pltpu.async_copy(col_h.at[pl.ds(ci * EC, EC)], col_v.at[b],
                         sem_i.at[b])

    def idx_wait(b):
        pltpu.make_async_copy(row_h.at[pl.ds(0, EC)], row_v.at[b],
                              sem_i.at[b]).wait()
        pltpu.make_async_copy(col_h.at[pl.ds(0, EC)], col_v.at[b],
                              sem_i.at[b]).wait()

    def g_issue(b):
        pltpu.async_copy(g_h.at[plsc.Indices(mcol.at[b], ignored_value=-1)],
                         rows_v.at[b], sem_g.at[b])

    def g_wait(b):
        pltpu.make_async_copy(
            g_h.at[plsc.Indices(mcol.at[b], ignored_value=-1)],
            rows_v.at[b], sem_g.at[b]).wait()

    def s_issue(b):
        pltpu.async_copy(rows_v.at[b],
                         acc.at[plsc.Indices(mrow.at[b], ignored_value=-1)],
                         sem_s.at[b], add=True)

    def s_wait(b):
        pltpu.make_async_copy(
            rows_v.at[b],
            acc.at[plsc.Indices(mrow.at[b], ignored_value=-1)],
            sem_s.at[b]).wait()

    for p in range(4):
        lo = (4 * c + p) * RNG
        pltpu.sync_copy(zacc, acc.at[pl.ds(s * (RNG // NS), RNG // NS)])
        plsc.subcore_barrier()

        def mask(b, lo=lo):
            for j in range(EC // 16):
                r = row_v[b, pl.ds(j * 16, 16)]
                cc = col_v[b, pl.ds(j * 16, 16)]
                m = (r >= lo) & (r < lo + RNG)
                mrow[b, pl.ds(j * 16, 16)] = jnp.where(m, r - lo, -1)
                mcol[b, pl.ds(j * 16, 16)] = jnp.where(m, cc, -1)

        idx_issue(0, start)
        idx_wait(0)
        mask(0)
        g_issue(0)
        idx_issue(1, start + 1)

        def body(k, carry, mask=mask):
            nB = start + 2 * k + 1
            nC = start + 2 * k + 2
            nD = start + 2 * k + 3
            vB = nB < start + cnt
            vC = nC < start + cnt
            vD = nD < start + cnt

            @pl.when(vB)
            def _():
                @pl.when(k > 0)
                def _():
                    s_wait(1)

                idx_wait(1)
                mask(1)
                g_issue(1)

            g_wait(0)
            s_issue(0)

            @pl.when(vC)
            def _():
                idx_issue(0, nC)

            @pl.when(vB)
            def _():
                g_wait(1)
                s_issue(1)

            @pl.when(vC)
            def _():
                s_wait(0)
                idx_wait(0)
                mask(0)
                g_issue(0)

            @pl.when(vD)
            def _():
                idx_issue(1, nD)

            return carry

        lax.fori_loop(0, (cnt + 1) // 2, body, 0)
        s_wait(0)
        s_wait(1)
        plsc.subcore_barrier()
        pltpu.sync_copy(acc.at[pl.ds(s * (RNG // NS), RNG // NS)],
                        out.at[pl.ds(lo + s * (RNG // NS), RNG // NS)])
        plsc.subcore_barrier()


_agg = functools.partial(
    pl.kernel, _agg_body,
    out_type=jax.ShapeDtypeStruct((NPAD, F), jnp.float32),
    mesh=plsc.VectorSubcoreMesh(**_MESH),
    scratch_types=[
        pltpu.VMEM((2, EC), jnp.int32),
        pltpu.VMEM((2, EC), jnp.int32),
        pltpu.VMEM((2, EC), jnp.int32),
        pltpu.VMEM((2, EC), jnp.int32),
        pltpu.VMEM((2, EC, F), jnp.float32),
        pltpu.VMEM_SHARED((RNG, F), jnp.float32),
        pltpu.SemaphoreType.DMA((2,)),
        pltpu.SemaphoreType.DMA((2,)),
        pltpu.SemaphoreType.DMA((2,)),
    ],
    compiler_params=_SC_PARAMS,
)()


GC = 400
NLCH = 2 * N_EDGES // GC


def _logits_body(src_h, dst_h, z_h, out, sidx, didx, zs, zd, tbuf, lbuf,
                 sem_i, sem_g):
    iot16 = lax.iota(jnp.int32, 16) * 16
    start, cnt = _share(NLCH, _wid(), NW)

    def idx_issue(b, ci):
        pltpu.async_copy(src_h.at[pl.ds(ci * GC, GC)], sidx.at[b],
                         sem_i.at[b])
        pltpu.async_copy(dst_h.at[pl.ds(ci * GC, GC)], didx.at[b],
                         sem_i.at[b])

    def idx_wait(b):
        pltpu.make_async_copy(src_h.at[pl.ds(0, GC)], sidx.at[b],
                              sem_i.at[b]).wait()
        pltpu.make_async_copy(dst_h.at[pl.ds(0, GC)], didx.at[b],
                              sem_i.at[b]).wait()

    def g_issue(b):
        pltpu.async_copy(z_h.at[sidx.at[b]], zs.at[b], sem_g.at[b])
        pltpu.async_copy(z_h.at[didx.at[b]], zd.at[b], sem_g.at[b])

    def g_wait(b):
        pltpu.make_async_copy(z_h.at[sidx.at[b]], zs.at[b],
                              sem_g.at[b]).wait()
        pltpu.make_async_copy(z_h.at[didx.at[b]], zd.at[b],
                              sem_g.at[b]).wait()

    def compute(b, ci):
        def group(g, carry2):
            for e in range(16):
                row = g * 16 + e
                acc = zs[b, row, pl.ds(0, 16)] * zd[b, row, pl.ds(0, 16)]
                for k in range(1, 4):
                    acc = acc + (zs[b, row, pl.ds(16 * k, 16)]
                                 * zd[b, row, pl.ds(16 * k, 16)])
                plsc.store_scatter(tbuf, [iot16 + e], acc)
            res = tbuf[pl.ds(0, 16)]
            for k in range(1, 16):
                res = res + tbuf[pl.ds(16 * k, 16)]
            lbuf[pl.ds(g * 16, 16)] = res
            return carry2

        lax.fori_loop(0, GC // 16, group, 0)
        pltpu.sync_copy(lbuf, out.at[pl.ds(ci * GC, GC)])

    idx_issue(0, start)
    idx_wait(0)
    g_issue(0)
    idx_issue(1, start + 1)

    def body(k, carry):
        nA = start + 2 * k
        nB = nA + 1
        nC = nA + 2
        nD = nA + 3
        vB = nB < start + cnt
        vC = nC < start + cnt
        vD = nD < start + cnt

        @pl.when(vB)
        def _():
            idx_wait(1)
            g_issue(1)

        g_wait(0)
        compute(0, nA)

        @pl.when(vC)
        def _():
            idx_issue(0, nC)

        @pl.when(vB)
        def _():
            g_wait(1)
            compute(1, nB)

        @pl.when(vC)
        def _():
            idx_wait(0)
            g_issue(0)

        @pl.when(vD)
        def _():
            idx_issue(1, nD)

        return carry

    lax.fori_loop(0, (cnt + 1) // 2, body, 0)


_logits = functools.partial(
    pl.kernel, _logits_body,
    out_type=jax.ShapeDtypeStruct((2 * N_EDGES,), jnp.float32),
    mesh=plsc.VectorSubcoreMesh(**_MESH),
    scratch_types=[
        pltpu.VMEM((2, GC), jnp.int32),
        pltpu.VMEM((2, GC), jnp.int32),
        pltpu.VMEM((2, GC, F), jnp.float32),
        pltpu.VMEM((2, GC, F), jnp.float32),
        pltpu.VMEM((256,), jnp.float32),
        pltpu.VMEM((GC,), jnp.float32),
        pltpu.SemaphoreType.DMA((2,)),
        pltpu.SemaphoreType.DMA((2,)),
    ],
    compiler_params=_SC_PARAMS,
)()


BN = 2048
GRID = NPAD // BN


def _mm1_body(da_ref, db_ref, x_ref, w_ref, b_ref, g_ref, dis_ref):
    deg = da_ref[...] + db_ref[...]
    dis = jnp.where(deg > 0, lax.rsqrt(deg), 0.0)
    h = jnp.dot(x_ref[...], w_ref[...],
                preferred_element_type=jnp.float32) + b_ref[...]
    g_ref[...] = dis[:, None] * h
    dis_ref[...] = dis


def _mm1(da, db, xp, w1, b1):
    return pl.pallas_call(
        _mm1_body,
        grid=(GRID,),
        in_specs=[
            pl.BlockSpec((BN,), lambda i: (i,)),
            pl.BlockSpec((BN,), lambda i: (i,)),
            pl.BlockSpec((BN, 128), lambda i: (i, 0)),
            pl.BlockSpec((128, F), lambda i: (0, 0)),
            pl.BlockSpec((1, F), lambda i: (0, 0)),
        ],
        out_specs=[
            pl.BlockSpec((BN, F), lambda i: (i, 0)),
            pl.BlockSpec((BN,), lambda i: (i,)),
        ],
        out_shape=[
            jax.ShapeDtypeStruct((NPAD, F), jnp.float32),
            jax.ShapeDtypeStruct((NPAD,), jnp.float32),
        ],
    )(da, db, xp, w1, b1)


def _mm2_body(dis_ref, a_ref, w_ref, b_ref, g_ref):
    dis = dis_ref[...]
    h1 = jnp.maximum(dis[:, None] * a_ref[...], 0.0)
    g_ref[...] = dis[:, None] * (
        jnp.dot(h1, w_ref[...], preferred_element_type=jnp.float32) + b_ref[...])


def _mm2(dis, a1, w2, b2):
    return pl.pallas_call(
        _mm2_body,
        grid=(GRID,),
        in_specs=[
            pl.BlockSpec((BN,), lambda i: (i,)),
            pl.BlockSpec((BN, F), lambda i: (i, 0)),
            pl.BlockSpec((F, F), lambda i: (0, 0)),
            pl.BlockSpec((1, F), lambda i: (0, 0)),
        ],
        out_specs=pl.BlockSpec((BN, F), lambda i: (i, 0)),
        out_shape=jax.ShapeDtypeStruct((NPAD, F), jnp.float32),
    )(dis, a1, w2, b2)


def _scale_body(dis_ref, a_ref, z_ref):
    z_ref[...] = dis_ref[...][:, None] * a_ref[...]


def _scale(dis, a2):
    return pl.pallas_call(
        _scale_body,
        grid=(GRID,),
        in_specs=[
            pl.BlockSpec((BN,), lambda i: (i,)),
            pl.BlockSpec((BN, F), lambda i: (i, 0)),
        ],
        out_specs=pl.BlockSpec((BN, F), lambda i: (i, 0)),
        out_shape=jax.ShapeDtypeStruct((NPAD, F), jnp.float32),
    )(dis, a2)


def kernel(x, pos_edge_index, neg_edge_index, W1, b1, W2, b2):
    pe = pos_edge_index.astype(jnp.int32)
    ne = neg_edge_index.astype(jnp.int32)
    row, col = pe[0], pe[1]
    xp = jnp.pad(x, ((0, NPAD - N_NODES), (0, 0)))
    zrow = jnp.zeros((NPAD // NS,), jnp.float32)
    zacc = jnp.zeros((RNG // NS, F), jnp.float32)

    deg2 = _bincount(row, zrow)
    g1, dis = _mm1(deg2[:NPAD], deg2[NPAD:], xp, W1, b1.reshape(1, F))
    a1 = _agg(row, col, g1, zacc)
    g2 = _mm2(dis, a1, W2, b2.reshape(1, F))
    a2 = _agg(row, col, g2, zacc)
    z = _scale(dis, a2)

    src = jnp.concatenate([pe[0], ne[0]])
    dst = jnp.concatenate([pe[1], ne[1]])
    return _logits(src, dst, z)


# speedup vs baseline: 17.6394x; 1.0652x over previous
import functools

import jax
import jax.numpy as jnp
from jax import lax
from jax.experimental import pallas as pl
from jax.experimental.pallas import tpu as pltpu
from jax.experimental.pallas import tpu_sc as plsc

N_NODES = 100000
N_EDGES = 3200000
F = 64
NPAD = 100352
RNG = NPAD // 8
NC, NS, L = 2, 16, 16
NW = NC * NS

_MESH = dict(core_axis_name="c", subcore_axis_name="s")
_SC_PARAMS = pltpu.CompilerParams(use_tc_tiling_on_sc=False,
                                  needs_layout_passes=False)


def _wid():
    return lax.axis_index("s") * NC + lax.axis_index("c")


def _share(total, idx, parts):
    q, rem = total // parts, total % parts
    start = idx * q + jnp.minimum(idx, rem)
    cnt = q + jnp.where(idx < rem, 1, 0)
    return start, cnt


BCC = 1024


def _bincount_body(row_h, zrow, out, idx_v, ones_v, acc):
    c = lax.axis_index("c")
    s = lax.axis_index("s")
    pltpu.sync_copy(zrow, acc.at[pl.ds(s * (NPAD // NS), NPAD // NS)])
    for t in range(BCC // 16):
        ones_v[pl.ds(16 * t, 16)] = jnp.full((16,), 1.0, jnp.float32)
    plsc.subcore_barrier()
    start, cnt = _share(N_EDGES // BCC, _wid(), NW)

    def chunk(ci, carry):
        pltpu.sync_copy(row_h.at[pl.ds(ci * BCC, BCC)], idx_v)
        pltpu.sync_copy(ones_v, acc.at[idx_v], add=True)
        return carry

    lax.fori_loop(start, start + cnt, chunk, 0)
    plsc.subcore_barrier()
    pltpu.sync_copy(acc.at[pl.ds(s * (NPAD // NS), NPAD // NS)],
                    out.at[pl.ds(c * NPAD + s * (NPAD // NS), NPAD // NS)])


_bincount = functools.partial(
    pl.kernel, _bincount_body,
    out_type=jax.ShapeDtypeStruct((2 * NPAD,), jnp.float32),
    mesh=plsc.VectorSubcoreMesh(**_MESH),
    scratch_types=[
        pltpu.VMEM((BCC,), jnp.int32),
        pltpu.VMEM((BCC,), jnp.float32),
        pltpu.VMEM_SHARED((NPAD,), jnp.float32),
    ],
    compiler_params=_SC_PARAMS,
)()


EC = 512
NCH = N_EDGES // EC


def _agg_body(row_h, col_h, g_h, zacc, out, row_v, col_v, mrow, mcol,
              rows_v, acc, sem_i, sem_g, sem_s):
    c = lax.axis_index("c")
    s = lax.axis_index("s")
    start, cnt = _share(NCH, s, NS)

    def idx_issue(b, ci):
        pltpu.async_copy(row_h.at[pl.ds(ci * EC, EC)], row_v.at[b],
                         sem_i.at[b])
        pltpu.async_copy(col_h.at[pl.ds(ci * EC, EC)], col_v.at[b],
                         sem_i.at[b])

    def idx_wait(b):
        pltpu.make_async_copy(row_h.at[pl.ds(0, EC)], row_v.at[b],
                              sem_i.at[b]).wait()
        pltpu.make_async_copy(col_h.at[pl.ds(0, EC)], col_v.at[b],
                              sem_i.at[b]).wait()

    def g_issue(b):
        pltpu.async_copy(g_h.at[plsc.Indices(mcol.at[b], ignored_value=-1)],
                         rows_v.at[b], sem_g.at[b])

    def g_wait(b):
        pltpu.make_async_copy(
            g_h.at[plsc.Indices(mcol.at[b], ignored_value=-1)],
            rows_v.at[b], sem_g.at[b]).wait()

    def s_issue(b):
        pltpu.async_copy(rows_v.at[b],
                         acc.at[plsc.Indices(mrow.at[b], ignored_value=-1)],
                         sem_s.at[b], add=True)

    def s_wait(b):
        pltpu.make_async_copy(
            rows_v.at[b],
            acc.at[plsc.Indices(mrow.at[b], ignored_value=-1)],
            sem_s.at[b]).wait()

    for p in range(4):
        lo = (4 * c + p) * RNG
        pltpu.sync_copy(zacc, acc.at[pl.ds(s * (RNG // NS), RNG // NS)])
        plsc.subcore_barrier()

        def mask(b, lo=lo):
            for j in range(EC // 16):
                r = row_v[b, pl.ds(j * 16, 16)]
                cc = col_v[b, pl.ds(j * 16, 16)]
                m = (r >= lo) & (r < lo + RNG)
                mrow[b, pl.ds(j * 16, 16)] = jnp.where(m, r - lo, -1)
                mcol[b, pl.ds(j * 16, 16)] = jnp.where(m, cc, -1)

        idx_issue(0, start)
        idx_wait(0)
        mask(0)
        g_issue(0)
        idx_issue(1, start + 1)

        def body(k, carry, mask=mask):
            nB = start + 2 * k + 1
            nC = start + 2 * k + 2
            nD = start + 2 * k + 3
            vB = nB < start + cnt
            vC = nC < start + cnt
            vD = nD < start + cnt

            @pl.when(vB)
            def _():
                @pl.when(k > 0)
                def _():
                    s_wait(1)

                idx_wait(1)
                mask(1)
                g_issue(1)

            g_wait(0)
            s_issue(0)

            @pl.when(vC)
            def _():
                idx_issue(0, nC)

            @pl.when(vB)
            def _():
                g_wait(1)
                s_issue(1)

            @pl.when(vC)
            def _():
                s_wait(0)
                idx_wait(0)
                mask(0)
                g_issue(0)

            @pl.when(vD)
            def _():
                idx_issue(1, nD)

            return carry

        lax.fori_loop(0, (cnt + 1) // 2, body, 0)
        s_wait(0)
        s_wait(1)
        plsc.subcore_barrier()
        pltpu.sync_copy(acc.at[pl.ds(s * (RNG // NS), RNG // NS)],
                        out.at[pl.ds(lo + s * (RNG // NS), RNG // NS)])
        plsc.subcore_barrier()


_agg = functools.partial(
    pl.kernel, _agg_body,
    out_type=jax.ShapeDtypeStruct((NPAD, F), jnp.float32),
    mesh=plsc.VectorSubcoreMesh(**_MESH),
    scratch_types=[
        pltpu.VMEM((2, EC), jnp.int32),
        pltpu.VMEM((2, EC), jnp.int32),
        pltpu.VMEM((2, EC), jnp.int32),
        pltpu.VMEM((2, EC), jnp.int32),
        pltpu.VMEM((2, EC, F), jnp.float32),
        pltpu.VMEM_SHARED((RNG, F), jnp.float32),
        pltpu.SemaphoreType.DMA((2,)),
        pltpu.SemaphoreType.DMA((2,)),
        pltpu.SemaphoreType.DMA((2,)),
    ],
    compiler_params=_SC_PARAMS,
)()


GC = 800
NLCH = 2 * N_EDGES // GC


def _logits_body(src_h, dst_h, z_h, out, sidx, didx, zs, zd, tbuf, lbuf,
                 sem_i, sem_g):
    iot16 = lax.iota(jnp.int32, 16) * 16
    start, cnt = _share(NLCH, _wid(), NW)

    def idx_issue(b, ci):
        pltpu.async_copy(src_h.at[pl.ds(ci * GC, GC)], sidx.at[b],
                         sem_i.at[b])
        pltpu.async_copy(dst_h.at[pl.ds(ci * GC, GC)], didx.at[b],
                         sem_i.at[b])

    def idx_wait(b):
        pltpu.make_async_copy(src_h.at[pl.ds(0, GC)], sidx.at[b],
                              sem_i.at[b]).wait()
        pltpu.make_async_copy(dst_h.at[pl.ds(0, GC)], didx.at[b],
                              sem_i.at[b]).wait()

    def g_issue(b):
        pltpu.async_copy(z_h.at[sidx.at[b]], zs.at[b], sem_g.at[b])
        pltpu.async_copy(z_h.at[didx.at[b]], zd.at[b], sem_g.at[b])

    def g_wait(b):
        pltpu.make_async_copy(z_h.at[sidx.at[b]], zs.at[b],
                              sem_g.at[b]).wait()
        pltpu.make_async_copy(z_h.at[didx.at[b]], zd.at[b],
                              sem_g.at[b]).wait()

    def compute(b, ci):
        def group(g, carry2):
            for e in range(16):
                row = g * 16 + e
                acc = None
                for k in range(2):
                    sp = zs[b, row, pl.ds(32 * k, 32)]
                    dp = zd[b, row, pl.ds(32 * k, 32)]
                    sa, sb = plsc.unpack(sp, format=plsc.PackFormat.INTERLEAVED)
                    da, db = plsc.unpack(dp, format=plsc.PackFormat.INTERLEAVED)
                    term = sa * da + sb * db
                    acc = term if acc is None else acc + term
                plsc.store_scatter(tbuf, [iot16 + e], acc)
            res = tbuf[pl.ds(0, 16)]
            for k in range(1, 16):
                res = res + tbuf[pl.ds(16 * k, 16)]
            lbuf[pl.ds(g * 16, 16)] = res
            return carry2

        lax.fori_loop(0, GC // 16, group, 0)
        pltpu.sync_copy(lbuf, out.at[pl.ds(ci * GC, GC)])

    idx_issue(0, start)
    idx_wait(0)
    g_issue(0)
    idx_issue(1, start + 1)

    def body(k, carry):
        nA = start + 2 * k
        nB = nA + 1
        nC = nA + 2
        nD = nA + 3
        vB = nB < start + cnt
        vC = nC < start + cnt
        vD = nD < start + cnt

        @pl.when(vB)
        def _():
            idx_wait(1)
            g_issue(1)

        g_wait(0)
        compute(0, nA)

        @pl.when(vC)
        def _():
            idx_issue(0, nC)

        @pl.when(vB)
        def _():
            g_wait(1)
            compute(1, nB)

        @pl.when(vC)
        def _():
            idx_wait(0)
            g_issue(0)

        @pl.when(vD)
        def _():
            idx_issue(1, nD)

        return carry

    lax.fori_loop(0, (cnt + 1) // 2, body, 0)


_logits = functools.partial(
    pl.kernel, _logits_body,
    out_type=jax.ShapeDtypeStruct((2 * N_EDGES,), jnp.float32),
    mesh=plsc.VectorSubcoreMesh(**_MESH),
    scratch_types=[
        pltpu.VMEM((2, GC), jnp.int32),
        pltpu.VMEM((2, GC), jnp.int32),
        pltpu.VMEM((2, GC, F), jnp.bfloat16),
        pltpu.VMEM((2, GC, F), jnp.bfloat16),
        pltpu.VMEM((256,), jnp.float32),
        pltpu.VMEM((GC,), jnp.float32),
        pltpu.SemaphoreType.DMA((2,)),
        pltpu.SemaphoreType.DMA((2,)),
    ],
    compiler_params=_SC_PARAMS,
)()


BN = 2048
GRID = NPAD // BN


def _mm1_body(da_ref, db_ref, x_ref, w_ref, b_ref, g_ref, dis_ref):
    deg = da_ref[...] + db_ref[...]
    dis = jnp.where(deg > 0, lax.rsqrt(deg), 0.0)
    h = jnp.dot(x_ref[...], w_ref[...],
                preferred_element_type=jnp.float32) + b_ref[...]
    g_ref[...] = dis[:, None] * h
    dis_ref[...] = dis


def _mm1(da, db, xp, w1, b1):
    return pl.pallas_call(
        _mm1_body,
        grid=(GRID,),
        in_specs=[
            pl.BlockSpec((BN,), lambda i: (i,)),
            pl.BlockSpec((BN,), lambda i: (i,)),
            pl.BlockSpec((BN, 128), lambda i: (i, 0)),
            pl.BlockSpec((128, F), lambda i: (0, 0)),
            pl.BlockSpec((1, F), lambda i: (0, 0)),
        ],
        out_specs=[
            pl.BlockSpec((BN, F), lambda i: (i, 0)),
            pl.BlockSpec((BN,), lambda i: (i,)),
        ],
        out_shape=[
            jax.ShapeDtypeStruct((NPAD, F), jnp.float32),
            jax.ShapeDtypeStruct((NPAD,), jnp.float32),
        ],
    )(da, db, xp, w1, b1)


def _mm2_body(dis_ref, a_ref, w_ref, b_ref, g_ref):
    dis = dis_ref[...]
    h1 = jnp.maximum(dis[:, None] * a_ref[...], 0.0)
    g_ref[...] = dis[:, None] * (
        jnp.dot(h1, w_ref[...], preferred_element_type=jnp.float32) + b_ref[...])


def _mm2(dis, a1, w2, b2):
    return pl.pallas_call(
        _mm2_body,
        grid=(GRID,),
        in_specs=[
            pl.BlockSpec((BN,), lambda i: (i,)),
            pl.BlockSpec((BN, F), lambda i: (i, 0)),
            pl.BlockSpec((F, F), lambda i: (0, 0)),
            pl.BlockSpec((1, F), lambda i: (0, 0)),
        ],
        out_specs=pl.BlockSpec((BN, F), lambda i: (i, 0)),
        out_shape=jax.ShapeDtypeStruct((NPAD, F), jnp.float32),
    )(dis, a1, w2, b2)


def _scale_body(dis_ref, a_ref, z_ref):
    z_ref[...] = (dis_ref[...][:, None] * a_ref[...]).astype(jnp.bfloat16)


def _scale(dis, a2):
    return pl.pallas_call(
        _scale_body,
        grid=(GRID,),
        in_specs=[
            pl.BlockSpec((BN,), lambda i: (i,)),
            pl.BlockSpec((BN, F), lambda i: (i, 0)),
        ],
        out_specs=pl.BlockSpec((BN, F), lambda i: (i, 0)),
        out_shape=jax.ShapeDtypeStruct((NPAD, F), jnp.bfloat16),
    )(dis, a2)


def kernel(x, pos_edge_index, neg_edge_index, W1, b1, W2, b2):
    pe = pos_edge_index.astype(jnp.int32)
    ne = neg_edge_index.astype(jnp.int32)
    row, col = pe[0], pe[1]
    xp = jnp.pad(x, ((0, NPAD - N_NODES), (0, 0)))
    zrow = jnp.zeros((NPAD // NS,), jnp.float32)
    zacc = jnp.zeros((RNG // NS, F), jnp.float32)

    deg2 = _bincount(row, zrow)
    g1, dis = _mm1(deg2[:NPAD], deg2[NPAD:], xp, W1, b1.reshape(1, F))
    a1 = _agg(row, col, g1, zacc)
    g2 = _mm2(dis, a1, W2, b2.reshape(1, F))
    a2 = _agg(row, col, g2, zacc)
    z = _scale(dis, a2)

    src = jnp.concatenate([pe[0], ne[0]])
    dst = jnp.concatenate([pe[1], ne[1]])
    return _logits(src, dst, z)


# speedup vs baseline: 29.1372x; 1.6518x over previous
import functools

import jax
import jax.numpy as jnp
from jax import lax
from jax.experimental import pallas as pl
from jax.experimental.pallas import tpu as pltpu
from jax.experimental.pallas import tpu_sc as plsc

N_NODES = 100000
N_EDGES = 3200000
F = 64
NPAD = 100352
RNG = NPAD // 8
NC, NS, L = 2, 16, 16
NW = NC * NS

_MESH = dict(core_axis_name="c", subcore_axis_name="s")
_SC_PARAMS = pltpu.CompilerParams(use_tc_tiling_on_sc=False,
                                  needs_layout_passes=False)


def _wid():
    return lax.axis_index("s") * NC + lax.axis_index("c")


def _share(total, idx, parts):
    q, rem = total // parts, total % parts
    start = idx * q + jnp.minimum(idx, rem)
    cnt = q + jnp.where(idx < rem, 1, 0)
    return start, cnt


BCC = 1024


def _bincount_body(row_h, zrow, out, idx_v, ones_v, acc):
    c = lax.axis_index("c")
    s = lax.axis_index("s")
    pltpu.sync_copy(zrow, acc.at[pl.ds(s * (NPAD // NS), NPAD // NS)])
    for t in range(BCC // 16):
        ones_v[pl.ds(16 * t, 16)] = jnp.full((16,), 1.0, jnp.float32)
    plsc.subcore_barrier()
    start, cnt = _share(N_EDGES // BCC, _wid(), NW)

    def chunk(ci, carry):
        pltpu.sync_copy(row_h.at[pl.ds(ci * BCC, BCC)], idx_v)
        pltpu.sync_copy(ones_v, acc.at[idx_v], add=True)
        return carry

    lax.fori_loop(start, start + cnt, chunk, 0)
    plsc.subcore_barrier()
    pltpu.sync_copy(acc.at[pl.ds(s * (NPAD // NS), NPAD // NS)],
                    out.at[pl.ds(c * NPAD + s * (NPAD // NS), NPAD // NS)])


_bincount = functools.partial(
    pl.kernel, _bincount_body,
    out_type=jax.ShapeDtypeStruct((2 * NPAD,), jnp.float32),
    mesh=plsc.VectorSubcoreMesh(**_MESH),
    scratch_types=[
        pltpu.VMEM((BCC,), jnp.int32),
        pltpu.VMEM((BCC,), jnp.float32),
        pltpu.VMEM_SHARED((NPAD,), jnp.float32),
    ],
    compiler_params=_SC_PARAMS,
)()


NWIN = 8
SEG = 512
NB_CAP = N_EDGES + SEG * NWIN * NW


def _count_body(row_h, out, row_v, bins, cbuf):
    w = _wid()
    iotav = lax.iota(jnp.int32, 16)
    ones = jnp.full((16,), 1, jnp.int32)
    for wi in range(NWIN):
        bins[wi, :] = jnp.zeros((16,), jnp.int32)

    for ch in range(25):
        pltpu.sync_copy(row_h.at[pl.ds((w * 25 + ch) * 4000, 4000)], row_v)

        def vreg(j, carry):
            r = row_v[pl.ds(j * 16, 16)]
            wv = lax.div(r, RNG)
            plsc.addupdate_scatter(bins, [wv, iotav], ones)
            return carry

        lax.fori_loop(0, 250, vreg, 0)

    for wi in range(NWIN):
        cbuf[pl.ds(wi * 16, 16)] = bins[wi, :]
    pltpu.sync_copy(cbuf, out.at[pl.ds(w * (NWIN * 16), NWIN * 16)])


_count = functools.partial(
    pl.kernel, _count_body,
    out_type=jax.ShapeDtypeStruct((NW * NWIN * 16,), jnp.int32),
    mesh=plsc.VectorSubcoreMesh(**_MESH),
    scratch_types=[
        pltpu.VMEM((4000,), jnp.int32),
        pltpu.VMEM((NWIN, 16), jnp.int32),
        pltpu.VMEM((NWIN * 16,), jnp.int32),
    ],
    compiler_params=_SC_PARAMS,
)()


def _seg_counts(binv, wid=None):
    base = jnp.int32(0)
    cur, at_list = [], []
    for w in range(NWIN):
        at = jnp.int32(0)
        mine = jnp.int32(0)
        for t in range(NW):
            c = jnp.sum(binv[pl.ds((t * NWIN + w) * 16, 16)])
            a = (c + (SEG - 1)) & (-SEG)
            at = at + a
            if wid is not None:
                mine = mine + jnp.where(jnp.int32(t) < wid, a, 0)
        if wid is not None:
            cur.append(base + mine)
        at_list.append(at)
        base = base + at
    return cur, at_list


def _bucket_body(row_h, col_h, cnt_h, rowb, colb, row_v, col_v, binv,
                 stR, stC):
    w = _wid()
    neg1 = jnp.full((16,), -1, jnp.int32)
    pltpu.sync_copy(cnt_h, binv)
    cur, _ = _seg_counts(binv, w)

    carry0 = tuple([jnp.int32(0)] * NWIN) + tuple(cur)

    def unit(args):
        u, ns, cs = args
        ns = list(ns)
        cs = list(cs)
        for j in range(10):
            off = u * 160 + j * 16
            r = row_v[pl.ds(off, 16)]
            cc = col_v[pl.ds(off, 16)]
            wv = lax.div(r, RNG)
            rl = r - wv * RNG
            for wi in range(NWIN):
                m = wv == wi
                cum = plsc.cumsum(m.astype(jnp.int32))
                idx = ns[wi] + cum - 1
                plsc.store_scatter(stR.at[wi], [idx], rl, mask=m)
                plsc.store_scatter(stC.at[wi], [idx], cc, mask=m)
                ns[wi] = ns[wi] + jnp.sum(m.astype(jnp.int32))
        for wi in range(NWIN):
            def fire(a, wi=wi):
                n, cu = a
                cua = pl.multiple_of(cu, SEG)
                pltpu.sync_copy(stR.at[wi, pl.ds(0, SEG)],
                                rowb.at[pl.ds(cua, SEG)])
                pltpu.sync_copy(stC.at[wi, pl.ds(0, SEG)],
                                colb.at[pl.ds(cua, SEG)])
                for k in range(11):
                    stR[wi, pl.ds(16 * k, 16)] = stR[wi,
                                                     pl.ds(SEG + 16 * k, 16)]
                    stC[wi, pl.ds(16 * k, 16)] = stC[wi,
                                                     pl.ds(SEG + 16 * k, 16)]
                return (n - SEG, cu + SEG)

            ns[wi], cs[wi] = lax.cond(ns[wi] >= SEG, fire, lambda a: a,
                                      (ns[wi], cs[wi]))
        return tuple(ns), tuple(cs)

    def chunk(ch, carry):
        ns, cs = carry[:NWIN], carry[NWIN:]
        pltpu.sync_copy(row_h.at[pl.ds(w * 100000 + ch * 4000, 4000)], row_v)
        pltpu.sync_copy(col_h.at[pl.ds(w * 100000 + ch * 4000, 4000)], col_v)

        def u_body(u, c2):
            ns2, cs2 = unit((u, c2[:NWIN], c2[NWIN:]))
            return ns2 + cs2

        return lax.fori_loop(0, 25, u_body, ns + cs)

    carry = lax.fori_loop(0, 25, chunk, carry0)
    ns, cs = carry[:NWIN], carry[NWIN:]

    for wi in range(NWIN):
        @pl.when(ns[wi] > 0)
        def _(wi=wi):
            n = ns[wi]
            iotav = lax.iota(jnp.int32, 16)
            for k in range(32):
                idx = n + 16 * k + iotav
                mm = idx < SEG
                plsc.store_scatter(stR.at[wi], [idx], neg1, mask=mm)
                plsc.store_scatter(stC.at[wi], [idx], neg1, mask=mm)
            cua = pl.multiple_of(cs[wi], SEG)
            pltpu.sync_copy(stR.at[wi, pl.ds(0, SEG)],
                            rowb.at[pl.ds(cua, SEG)])
            pltpu.sync_copy(stC.at[wi, pl.ds(0, SEG)],
                            colb.at[pl.ds(cua, SEG)])


_bucket = functools.partial(
    pl.kernel, _bucket_body,
    out_type=[jax.ShapeDtypeStruct((NB_CAP,), jnp.int32),
              jax.ShapeDtypeStruct((NB_CAP,), jnp.int32)],
    mesh=plsc.VectorSubcoreMesh(**_MESH),
    scratch_types=[
        pltpu.VMEM((4000,), jnp.int32),
        pltpu.VMEM((4000,), jnp.int32),
        pltpu.VMEM((NW * NWIN * 16,), jnp.int32),
        pltpu.VMEM((NWIN, 704), jnp.int32),
        pltpu.VMEM((NWIN, 704), jnp.int32),
    ],
    compiler_params=_SC_PARAMS,
)()


EC = 512


def _agg_body(row_h, col_h, cnt_h, g_h, zacc, out, row_v, col_v, mrow, mcol,
              rows_v, acc, binv, sem_i, sem_g, sem_s):
    c = lax.axis_index("c")
    s = lax.axis_index("s")
    pltpu.sync_copy(cnt_h, binv)
    _, at_list = _seg_counts(binv)
    seg_start = []
    base = jnp.int32(0)
    for wi in range(NWIN):
        seg_start.append(base)
        base = base + lax.div(at_list[wi], EC)

    def idx_issue(b, ci):
        pltpu.async_copy(row_h.at[pl.ds(ci * EC, EC)], row_v.at[b],
                         sem_i.at[b])
        pltpu.async_copy(col_h.at[pl.ds(ci * EC, EC)], col_v.at[b],
                         sem_i.at[b])

    def idx_wait(b):
        pltpu.make_async_copy(row_h.at[pl.ds(0, EC)], row_v.at[b],
                              sem_i.at[b]).wait()
        pltpu.make_async_copy(col_h.at[pl.ds(0, EC)], col_v.at[b],
                              sem_i.at[b]).wait()

    def g_issue(b):
        pltpu.async_copy(g_h.at[plsc.Indices(mcol.at[b], ignored_value=-1)],
                         rows_v.at[b], sem_g.at[b])

    def g_wait(b):
        pltpu.make_async_copy(
            g_h.at[plsc.Indices(mcol.at[b], ignored_value=-1)],
            rows_v.at[b], sem_g.at[b]).wait()

    def s_issue(b):
        pltpu.async_copy(rows_v.at[b],
                         acc.at[plsc.Indices(mrow.at[b], ignored_value=-1)],
                         sem_s.at[b], add=True)

    def s_wait(b):
        pltpu.make_async_copy(
            rows_v.at[b],
            acc.at[plsc.Indices(mrow.at[b], ignored_value=-1)],
            sem_s.at[b]).wait()

    def mask(b):
        for j in range(EC // 16):
            mrow[b, pl.ds(j * 16, 16)] = row_v[b, pl.ds(j * 16, 16)]
            mcol[b, pl.ds(j * 16, 16)] = col_v[b, pl.ds(j * 16, 16)]

    for p in range(4):
        w = 4 * c + p
        lo = w * RNG
        pltpu.sync_copy(zacc, acc.at[pl.ds(s * (RNG // NS), RNG // NS)])
        plsc.subcore_barrier()
        wstart = jnp.int32(0)
        wlen = jnp.int32(0)
        for wi in range(NWIN):
            sel = jnp.int32(wi) == w
            wstart = wstart + jnp.where(sel, seg_start[wi], 0)
            wlen = wlen + jnp.where(sel, lax.div(at_list[wi], EC), 0)
        start, cnt = _share(wlen, s, NS)
        start = start + wstart

        @pl.when(cnt > 0)
        def _():
            idx_issue(0, start)
            idx_wait(0)
            mask(0)
            g_issue(0)

        @pl.when(cnt > 1)
        def _():
            idx_issue(1, start + 1)

        def body(k, carry):
            nB = start + 2 * k + 1
            nC = start + 2 * k + 2
            nD = start + 2 * k + 3
            vB = nB < start + cnt
            vC = nC < start + cnt
            vD = nD < start + cnt

            @pl.when(vB)
            def _():
                @pl.when(k > 0)
                def _():
                    s_wait(1)

                idx_wait(1)
                mask(1)
                g_issue(1)

            g_wait(0)
            s_issue(0)

            @pl.when(vC)
            def _():
                idx_issue(0, nC)

            @pl.when(vB)
            def _():
                g_wait(1)
                s_issue(1)

            @pl.when(vC)
            def _():
                s_wait(0)
                idx_wait(0)
                mask(0)
                g_issue(0)

            @pl.when(vD)
            def _():
                idx_issue(1, nD)

            return carry

        lax.fori_loop(0, (cnt + 1) // 2, body, 0)

        @pl.when(cnt > 0)
        def _():
            s_wait(0)

        @pl.when(cnt > 1)
        def _():
            s_wait(1)

        plsc.subcore_barrier()
        pltpu.sync_copy(acc.at[pl.ds(s * (RNG // NS), RNG // NS)],
                        out.at[pl.ds(lo + s * (RNG // NS), RNG // NS)])
        plsc.subcore_barrier()


_agg = functools.partial(
    pl.kernel, _agg_body,
    out_type=jax.ShapeDtypeStruct((NPAD, F), jnp.float32),
    mesh=plsc.VectorSubcoreMesh(**_MESH),
    scratch_types=[
        pltpu.VMEM((2, EC), jnp.int32),
        pltpu.VMEM((2, EC), jnp.int32),
        pltpu.VMEM((2, EC), jnp.int32),
        pltpu.VMEM((2, EC), jnp.int32),
        pltpu.VMEM((2, EC, F), jnp.float32),
        pltpu.VMEM_SHARED((RNG, F), jnp.float32),
        pltpu.VMEM((NW * NWIN * 16,), jnp.int32),
        pltpu.SemaphoreType.DMA((2,)),
        pltpu.SemaphoreType.DMA((2,)),
        pltpu.SemaphoreType.DMA((2,)),
    ],
    compiler_params=_SC_PARAMS,
)()


GC = 800
NLCH = 2 * N_EDGES // GC


def _logits_body(src_h, dst_h, z_h, out, sidx, didx, zs, zd, tbuf, lbuf,
                 sem_i, sem_g):
    iot16 = lax.iota(jnp.int32, 16) * 16
    start, cnt = _share(NLCH, _wid(), NW)

    def idx_issue(b, ci):
        pltpu.async_copy(src_h.at[pl.ds(ci * GC, GC)], sidx.at[b],
                         sem_i.at[b])
        pltpu.async_copy(dst_h.at[pl.ds(ci * GC, GC)], didx.at[b],
                         sem_i.at[b])

    def idx_wait(b):
        pltpu.make_async_copy(src_h.at[pl.ds(0, GC)], sidx.at[b],
                              sem_i.at[b]).wait()
        pltpu.make_async_copy(dst_h.at[pl.ds(0, GC)], didx.at[b],
                              sem_i.at[b]).wait()

    def g_issue(b):
        pltpu.async_copy(z_h.at[sidx.at[b]], zs.at[b], sem_g.at[b])
        pltpu.async_copy(z_h.at[didx.at[b]], zd.at[b], sem_g.at[b])

    def g_wait(b):
        pltpu.make_async_copy(z_h.at[sidx.at[b]], zs.at[b],
                              sem_g.at[b]).wait()
        pltpu.make_async_copy(z_h.at[didx.at[b]], zd.at[b],
                              sem_g.at[b]).wait()

    def compute(b, ci):
        def group(g, carry2):
            for e in range(16):
                row = g * 16 + e
                acc = None
                for k in range(2):
                    sp = zs[b, row, pl.ds(32 * k, 32)]
                    dp = zd[b, row, pl.ds(32 * k, 32)]
                    sa, sb = plsc.unpack(sp, format=plsc.PackFormat.INTERLEAVED)
                    da, db = plsc.unpack(dp, format=plsc.PackFormat.INTERLEAVED)
                    term = sa * da + sb * db
                    acc = term if acc is None else acc + term
                plsc.store_scatter(tbuf, [iot16 + e], acc)
            res = tbuf[pl.ds(0, 16)]
            for k in range(1, 16):
                res = res + tbuf[pl.ds(16 * k, 16)]
            lbuf[pl.ds(g * 16, 16)] = res
            return carry2

        lax.fori_loop(0, GC // 16, group, 0)
        pltpu.sync_copy(lbuf, out.at[pl.ds(ci * GC, GC)])

    idx_issue(0, start)
    idx_wait(0)
    g_issue(0)
    idx_issue(1, start + 1)

    def body(k, carry):
        nA = start + 2 * k
        nB = nA + 1
        nC = nA + 2
        nD = nA + 3
        vB = nB < start + cnt
        vC = nC < start + cnt
        vD = nD < start + cnt

        @pl.when(vB)
        def _():
            idx_wait(1)
            g_issue(1)

        g_wait(0)
        compute(0, nA)

        @pl.when(vC)
        def _():
            idx_issue(0, nC)

        @pl.when(vB)
        def _():
            g_wait(1)
            compute(1, nB)

        @pl.when(vC)
        def _():
            idx_wait(0)
            g_issue(0)

        @pl.when(vD)
        def _():
            idx_issue(1, nD)

        return carry

    lax.fori_loop(0, (cnt + 1) // 2, body, 0)


_logits = functools.partial(
    pl.kernel, _logits_body,
    out_type=jax.ShapeDtypeStruct((2 * N_EDGES,), jnp.float32),
    mesh=plsc.VectorSubcoreMesh(**_MESH),
    scratch_types=[
        pltpu.VMEM((2, GC), jnp.int32),
        pltpu.VMEM((2, GC), jnp.int32),
        pltpu.VMEM((2, GC, F), jnp.bfloat16),
        pltpu.VMEM((2, GC, F), jnp.bfloat16),
        pltpu.VMEM((256,), jnp.float32),
        pltpu.VMEM((GC,), jnp.float32),
        pltpu.SemaphoreType.DMA((2,)),
        pltpu.SemaphoreType.DMA((2,)),
    ],
    compiler_params=_SC_PARAMS,
)()


BN = 2048
GRID = NPAD // BN


def _mm1_body(da_ref, db_ref, x_ref, w_ref, b_ref, g_ref, dis_ref):
    deg = da_ref[...] + db_ref[...]
    dis = jnp.where(deg > 0, lax.rsqrt(deg), 0.0)
    h = jnp.dot(x_ref[...], w_ref[...],
                preferred_element_type=jnp.float32) + b_ref[...]
    g_ref[...] = dis[:, None] * h
    dis_ref[...] = dis


def _mm1(da, db, xp, w1, b1):
    return pl.pallas_call(
        _mm1_body,
        grid=(GRID,),
        in_specs=[
            pl.BlockSpec((BN,), lambda i: (i,)),
            pl.BlockSpec((BN,), lambda i: (i,)),
            pl.BlockSpec((BN, 128), lambda i: (i, 0)),
            pl.BlockSpec((128, F), lambda i: (0, 0)),
            pl.BlockSpec((1, F), lambda i: (0, 0)),
        ],
        out_specs=[
            pl.BlockSpec((BN, F), lambda i: (i, 0)),
            pl.BlockSpec((BN,), lambda i: (i,)),
        ],
        out_shape=[
            jax.ShapeDtypeStruct((NPAD, F), jnp.float32),
            jax.ShapeDtypeStruct((NPAD,), jnp.float32),
        ],
    )(da, db, xp, w1, b1)


def _mm2_body(dis_ref, a_ref, w_ref, b_ref, g_ref):
    dis = dis_ref[...]
    h1 = jnp.maximum(dis[:, None] * a_ref[...], 0.0)
    g_ref[...] = dis[:, None] * (
        jnp.dot(h1, w_ref[...], preferred_element_type=jnp.float32) + b_ref[...])


def _mm2(dis, a1, w2, b2):
    return pl.pallas_call(
        _mm2_body,
        grid=(GRID,),
        in_specs=[
            pl.BlockSpec((BN,), lambda i: (i,)),
            pl.BlockSpec((BN, F), lambda i: (i, 0)),
            pl.BlockSpec((F, F), lambda i: (0, 0)),
            pl.BlockSpec((1, F), lambda i: (0, 0)),
        ],
        out_specs=pl.BlockSpec((BN, F), lambda i: (i, 0)),
        out_shape=jax.ShapeDtypeStruct((NPAD, F), jnp.float32),
    )(dis, a1, w2, b2)


def _scale_body(dis_ref, a_ref, z_ref):
    z_ref[...] = (dis_ref[...][:, None] * a_ref[...]).astype(jnp.bfloat16)


def _scale(dis, a2):
    return pl.pallas_call(
        _scale_body,
        grid=(GRID,),
        in_specs=[
            pl.BlockSpec((BN,), lambda i: (i,)),
            pl.BlockSpec((BN, F), lambda i: (i, 0)),
        ],
        out_specs=pl.BlockSpec((BN, F), lambda i: (i, 0)),
        out_shape=jax.ShapeDtypeStruct((NPAD, F), jnp.bfloat16),
    )(dis, a2)


def kernel(x, pos_edge_index, neg_edge_index, W1, b1, W2, b2):
    pe = pos_edge_index.astype(jnp.int32)
    ne = neg_edge_index.astype(jnp.int32)
    row, col = pe[0], pe[1]
    xp = jnp.pad(x, ((0, NPAD - N_NODES), (0, 0)))
    zrow = jnp.zeros((NPAD // NS,), jnp.float32)
    zacc = jnp.zeros((RNG // NS, F), jnp.float32)

    deg2 = _bincount(row, zrow)
    cnts = _count(row)
    rowb, colb = _bucket(row, col, cnts)
    g1, dis = _mm1(deg2[:NPAD], deg2[NPAD:], xp, W1, b1.reshape(1, F))
    a1 = _agg(rowb, colb, cnts, g1, zacc)
    g2 = _mm2(dis, a1, W2, b2.reshape(1, F))
    a2 = _agg(rowb, colb, cnts, g2, zacc)
    z = _scale(dis, a2)

    src = jnp.concatenate([pe[0], ne[0]])
    dst = jnp.concatenate([pe[1], ne[1]])
    return _logits(src, dst, z)
